# baseline (speedup 1.0000x reference)
import functools

import jax
import jax.numpy as jnp
from jax import lax
from jax.experimental import pallas as pl

M_MAX = 8
PHI = 1.6
JITTER = 1e-4


def _gate_up_kernel(s_ref, geom_ref, mask_ref, emb_ref, ln1g_ref, ln1b_ref,
                    w1_ref, b1_ref, w2_ref, b2_ref,
                    s_child_ref, geom_child_ref, g_ref, *, kb, m):
    s_blk = s_ref[...]
    C = s_blk.shape[1]
    rows = kb * m

    s_rep = jnp.broadcast_to(s_blk[:, None, :], (kb, m, C)).reshape(rows, C)
    e_rep = jnp.broadcast_to(emb_ref[...][None, :, :], (kb, m, C)).reshape(rows, C)

    gate_in = s_rep + e_rep
    mu = jnp.mean(gate_in, axis=-1, keepdims=True)
    var = jnp.mean(jnp.square(gate_in - mu), axis=-1, keepdims=True)
    h = (gate_in - mu) * lax.rsqrt(var + 1e-5)
    h = h * ln1g_ref[...] + ln1b_ref[...]

    h1 = jnp.dot(h, w1_ref[...], preferred_element_type=jnp.float32) + b1_ref[...]
    h1 = h1 * jax.nn.sigmoid(h1)
    bg = jnp.dot(h1, w2_ref[...], preferred_element_type=jnp.float32) + b2_ref[...]

    m_rep = jnp.broadcast_to(mask_ref[...][:, None, :], (kb, m, 1)).reshape(rows, 1)
    g = jax.nn.sigmoid(bg) * m_rep

    s_child_ref[...] = g * s_rep
    g_ref[...] = jnp.transpose(g, (1, 0))[None]

    geom_blk = geom_ref[...]
    geom_rep = jnp.broadcast_to(geom_blk[:, None, :], (kb, m, 12)).reshape(rows, 12)
    idx = lax.broadcasted_iota(jnp.int32, (1, 12), 1)
    scale = jnp.where(idx < 3, 1.0, PHI ** -2).astype(jnp.float32)
    shift = jnp.where((idx == 3) | (idx == 7) | (idx == 11),
                      JITTER, 0.0).astype(jnp.float32)
    geom_child_ref[...] = geom_rep * scale + shift


@jax.jit
def kernel(s_parent, mu_p, Sigma_p, mask_parent, xi_noise, params):
    B, Kp, C = s_parent.shape
    M = M_MAX
    Kcand = Kp * M
    N = B * Kp
    KB = 512
    NBLK = N // KB
    rows = KB * M

    s2 = s_parent.reshape(N, C)
    geom = jnp.concatenate(
        [mu_p.reshape(N, 3), Sigma_p.reshape(N, 9)], axis=-1)
    mask2 = mask_parent.reshape(N, 1)

    p = params
    emb = p['embed_w']
    ln1g = p['ln1_g'].reshape(1, C)
    ln1b = p['ln1_b'].reshape(1, C)
    b1 = p['b1'].reshape(1, C)
    b2 = p['b2'].reshape(1, 1)

    kfn = functools.partial(_gate_up_kernel, kb=KB, m=M)
    out_shapes = (
        jax.ShapeDtypeStruct((N * M, C), jnp.float32),
        jax.ShapeDtypeStruct((N * M, 12), jnp.float32),
        jax.ShapeDtypeStruct((NBLK, 1, rows), jnp.float32),
    )
    in_specs = [
        pl.BlockSpec((KB, C), lambda i: (i, 0)),
        pl.BlockSpec((KB, 12), lambda i: (i, 0)),
        pl.BlockSpec((KB, 1), lambda i: (i, 0)),
        pl.BlockSpec((M, C), lambda i: (0, 0)),
        pl.BlockSpec((1, C), lambda i: (0, 0)),
        pl.BlockSpec((1, C), lambda i: (0, 0)),
        pl.BlockSpec((C, C), lambda i: (0, 0)),
        pl.BlockSpec((1, C), lambda i: (0, 0)),
        pl.BlockSpec((C, 1), lambda i: (0, 0)),
        pl.BlockSpec((1, 1), lambda i: (0, 0)),
    ]
    out_specs = (
        pl.BlockSpec((rows, C), lambda i: (i, 0)),
        pl.BlockSpec((rows, 12), lambda i: (i, 0)),
        pl.BlockSpec((1, 1, rows), lambda i: (i, 0, 0)),
    )
    s_child, geom_child, g = pl.pallas_call(
        kfn,
        grid=(NBLK,),
        in_specs=in_specs,
        out_specs=out_specs,
        out_shape=out_shapes,
    )(s2, geom, mask2, emb, ln1g, ln1b, p['W1'], b1, p['W2'], b2)

    s_child0 = s_child.reshape(B, Kcand, C)
    geom_child = geom_child.reshape(B, Kcand, 12)
    mu_child = geom_child[..., :3]
    Sigma_child = geom_child[..., 3:].reshape(B, Kcand, 3, 3)
    g = g.reshape(B, Kcand)
    loss_count = jnp.zeros((), jnp.float32)
    return (s_child0, mu_child, Sigma_child, g, loss_count)


# speedup vs baseline: 1.0282x; 1.0282x over previous
import functools

import jax
import jax.numpy as jnp
from jax import lax
from jax.experimental import pallas as pl

M_MAX = 8
PHI = 1.6
JITTER = 1e-4


def _gate_up_kernel(s_ref, geom_ref, mask_ref, emb_ref, ln1g_ref, ln1b_ref,
                    w1_ref, b1_ref, w2_ref, b2_ref,
                    s_child_ref, geom_child_ref, g_ref, mu_child_ref,
                    *, kb, m):
    s_blk = s_ref[...]
    C = s_blk.shape[1]
    rows = kb * m

    s_rep = jnp.broadcast_to(s_blk[:, None, :], (kb, m, C)).reshape(rows, C)
    e_rep = jnp.broadcast_to(emb_ref[...][None, :, :], (kb, m, C)).reshape(rows, C)

    gate_in = s_rep + e_rep
    mu = jnp.mean(gate_in, axis=-1, keepdims=True)
    var = jnp.mean(jnp.square(gate_in - mu), axis=-1, keepdims=True)
    h = (gate_in - mu) * lax.rsqrt(var + 1e-5)
    h = h * ln1g_ref[...] + ln1b_ref[...]

    h1 = jnp.dot(h, w1_ref[...], preferred_element_type=jnp.float32) + b1_ref[...]
    h1 = h1 * jax.nn.sigmoid(h1)
    bg = jnp.dot(h1, w2_ref[...], preferred_element_type=jnp.float32) + b2_ref[...]

    m_rep = jnp.broadcast_to(mask_ref[...][:, None, :], (kb, m, 1)).reshape(rows, 1)
    g = jax.nn.sigmoid(bg) * m_rep

    s_child_ref[...] = g * s_rep
    g_ref[...] = jnp.transpose(g, (1, 0))[None]

    geom_blk = geom_ref[...]
    geom_rep = jnp.broadcast_to(geom_blk[:, None, :], (kb, m, 12)).reshape(rows, 12)
    idx = lax.broadcasted_iota(jnp.int32, (1, 12), 1)
    scale = jnp.where(idx < 3, 1.0, PHI ** -2).astype(jnp.float32)
    shift = jnp.where((idx == 3) | (idx == 7) | (idx == 11),
                      JITTER, 0.0).astype(jnp.float32)
    geom_child_ref[...] = geom_rep * scale + shift
    mu_child_ref[...] = geom_rep[:, :3]


@jax.jit
def kernel(s_parent, mu_p, Sigma_p, mask_parent, xi_noise, params):
    B, Kp, C = s_parent.shape
    M = M_MAX
    Kcand = Kp * M
    N = B * Kp
    KB = 256
    NBLK = N // KB
    rows = KB * M

    s2 = s_parent.reshape(N, C)
    geom = jnp.concatenate(
        [mu_p.reshape(N, 3), Sigma_p.reshape(N, 9)], axis=-1)
    mask2 = mask_parent.reshape(N, 1)

    p = params
    emb = p['embed_w']
    ln1g = p['ln1_g'].reshape(1, C)
    ln1b = p['ln1_b'].reshape(1, C)
    b1 = p['b1'].reshape(1, C)
    b2 = p['b2'].reshape(1, 1)

    kfn = functools.partial(_gate_up_kernel, kb=KB, m=M)
    out_shapes = (
        jax.ShapeDtypeStruct((N * M, C), jnp.float32),
        jax.ShapeDtypeStruct((N * M, 12), jnp.float32),
        jax.ShapeDtypeStruct((NBLK, 1, rows), jnp.float32),
        jax.ShapeDtypeStruct((N * M, 3), jnp.float32),
    )
    in_specs = [
        pl.BlockSpec((KB, C), lambda i: (i, 0)),
        pl.BlockSpec((KB, 12), lambda i: (i, 0)),
        pl.BlockSpec((KB, 1), lambda i: (i, 0)),
        pl.BlockSpec((M, C), lambda i: (0, 0)),
        pl.BlockSpec((1, C), lambda i: (0, 0)),
        pl.BlockSpec((1, C), lambda i: (0, 0)),
        pl.BlockSpec((C, C), lambda i: (0, 0)),
        pl.BlockSpec((1, C), lambda i: (0, 0)),
        pl.BlockSpec((C, 1), lambda i: (0, 0)),
        pl.BlockSpec((1, 1), lambda i: (0, 0)),
    ]
    out_specs = (
        pl.BlockSpec((rows, C), lambda i: (i, 0)),
        pl.BlockSpec((rows, 12), lambda i: (i, 0)),
        pl.BlockSpec((1, 1, rows), lambda i: (i, 0, 0)),
        pl.BlockSpec((rows, 3), lambda i: (i, 0)),
    )
    s_child, geom_child, g, mu_dir = pl.pallas_call(
        kfn,
        grid=(NBLK,),
        in_specs=in_specs,
        out_specs=out_specs,
        out_shape=out_shapes,
    )(s2, geom, mask2, emb, ln1g, ln1b, p['W1'], b1, p['W2'], b2)

    s_child0 = s_child.reshape(B, Kcand, C)
    geom_child = geom_child.reshape(B, Kcand, 12)
    mu_child = mu_dir.reshape(B, Kcand, 3)
    Sigma_child = geom_child[..., 3:].reshape(B, Kcand, 3, 3)
    g = g.reshape(B, Kcand)
    loss_count = jnp.zeros((), jnp.float32)
    return (s_child0, mu_child, Sigma_child, g, loss_count)
